# baseline (speedup 1.0000x reference)
import functools

import jax
import jax.numpy as jnp
from jax import lax
from jax.experimental import pallas as pl
from jax.experimental.pallas import tpu as pltpu
from jax.experimental.pallas import tpu_sc as plsc

N = 10000
E = 320000
D = 128

NC = 2
NS = 16
NW = NC * NS
C = 128
STEPS0 = 36
STEPS1 = 124
EP0 = STEPS0 * C
EP1 = STEPS1 * C
E_PAD = (EP0 + EP1) * NS
NPAD = 10240
RT = NPAD // NS
RT_LAST = N - (NS - 1) * RT

_mesh = plsc.VectorSubcoreMesh(core_axis_name="c", subcore_axis_name="s")


@functools.partial(
    pl.kernel,
    out_type=jax.ShapeDtypeStruct((NC, NPAD, D), jnp.float32),
    mesh=_mesh,
    scratch_types=[
        pltpu.VMEM((max(EP0, EP1),), jnp.int32),
        pltpu.VMEM((C,), jnp.int32),
        pltpu.VMEM((C,), jnp.int32),
        pltpu.VMEM((C, D), jnp.float32),
        pltpu.VMEM((C, D), jnp.float32),
        pltpu.VMEM_SHARED((NPAD, D), jnp.float32),
        pltpu.SemaphoreType.DMA,
        pltpu.SemaphoreType.DMA,
    ],
)
def _segment_sum(h_hbm, zeros_hbm, src_hbm, dst_hbm, out_hbm,
                 src_all, dst_v0, dst_v1, rows_v0, rows_v1, acc_sh,
                 sem0, sem1):
    cid = lax.axis_index("c")
    sid = lax.axis_index("s")
    last = sid == NS - 1
    r0 = sid * RT

    @pl.when(jnp.logical_and(cid == 0, jnp.logical_not(last)))
    def _():
        pltpu.sync_copy(h_hbm.at[pl.ds(r0, RT)], acc_sh.at[pl.ds(r0, RT)])

    @pl.when(jnp.logical_and(cid == 0, last))
    def _():
        pltpu.sync_copy(h_hbm.at[pl.ds(r0, RT_LAST)],
                        acc_sh.at[pl.ds(r0, RT_LAST)])

    @pl.when(jnp.logical_and(cid != 0, jnp.logical_not(last)))
    def _():
        pltpu.sync_copy(zeros_hbm.at[pl.ds(r0, RT)], acc_sh.at[pl.ds(r0, RT)])

    @pl.when(jnp.logical_and(cid != 0, last))
    def _():
        pltpu.sync_copy(zeros_hbm.at[pl.ds(r0, RT_LAST)],
                        acc_sh.at[pl.ds(r0, RT_LAST)])

    plsc.subcore_barrier()

    steps = jnp.where(cid == 0, STEPS0, STEPS1)
    base = jnp.where(cid == 0, sid * EP0, NS * EP0 + sid * EP1)
    dst_bufs = (dst_v0, dst_v1)
    row_bufs = (rows_v0, rows_v1)
    sems = (sem0, sem1)

    @pl.when(cid == 0)
    def _():
        pltpu.sync_copy(src_hbm.at[pl.ds(base, EP0)],
                        src_all.at[pl.ds(0, EP0)])

    @pl.when(cid != 0)
    def _():
        pltpu.sync_copy(src_hbm.at[pl.ds(base, EP1)],
                        src_all.at[pl.ds(0, EP1)])

    pltpu.async_copy(h_hbm.at[src_all.at[pl.ds(0, C)]], rows_v0, sem0)
    pltpu.async_copy(h_hbm.at[src_all.at[pl.ds(C, C)]], rows_v1, sem1)

    def body(k, carry):
        for t in range(2):
            j = 2 * k + t
            db, rb, sm = dst_bufs[t], row_bufs[t], sems[t]
            pltpu.make_async_copy(h_hbm.at[src_all.at[pl.ds(j * C, C)]],
                                  rb, sm).wait()
            pltpu.sync_copy(dst_hbm.at[pl.ds(base + j * C, C)], db)
            pltpu.sync_copy(rb, acc_sh.at[db], add=True)

            @pl.when(j + 2 < steps)
            def _():
                pltpu.async_copy(
                    h_hbm.at[src_all.at[pl.ds((j + 2) * C, C)]], rb, sm)
        return carry

    lax.fori_loop(0, steps // 2, body, 0)

    plsc.subcore_barrier()

    @pl.when(jnp.logical_not(last))
    def _():
        pltpu.sync_copy(acc_sh.at[pl.ds(r0, RT)],
                        out_hbm.at[cid, pl.ds(r0, RT)])

    @pl.when(last)
    def _():
        pltpu.sync_copy(acc_sh.at[pl.ds(r0, RT_LAST)],
                        out_hbm.at[cid, pl.ds(r0, RT_LAST)])


BN = 1000


def _mlp_body(p0_ref, p1_ref, wa_ref, ba_ref, wb_ref, bb_ref, o_ref):
    h = p0_ref[...] + p1_ref[...]
    a = jnp.maximum(
        jnp.dot(h, wa_ref[...], preferred_element_type=jnp.float32)
        + ba_ref[...], 0.0)
    o = jnp.maximum(
        jnp.dot(a, wb_ref[...], preferred_element_type=jnp.float32)
        + bb_ref[...], 0.0)
    o_ref[...] = o


def _mlp(p0, p1, Wa, ba, Wb, bb):
    return pl.pallas_call(
        _mlp_body,
        grid=(N // BN,),
        in_specs=[
            pl.BlockSpec((BN, D), lambda i: (i, 0)),
            pl.BlockSpec((BN, D), lambda i: (i, 0)),
            pl.BlockSpec((D, D), lambda i: (0, 0)),
            pl.BlockSpec((1, D), lambda i: (0, 0)),
            pl.BlockSpec((D, D), lambda i: (0, 0)),
            pl.BlockSpec((1, D), lambda i: (0, 0)),
        ],
        out_specs=pl.BlockSpec((BN, D), lambda i: (i, 0)),
        out_shape=jax.ShapeDtypeStruct((N, D), jnp.float32),
    )(p0, p1, Wa, ba.reshape(1, D), Wb, bb.reshape(1, D))


def kernel(x, edge_index, W1a, b1a, W1b, b1b, W2a, b2a, W2b, b2b):
    src = edge_index[0].astype(jnp.int32)
    dst = edge_index[1].astype(jnp.int32)
    pad = E_PAD - E
    src_p = jnp.concatenate([src, jnp.zeros((pad,), jnp.int32)])
    pad_dst = N + (jnp.arange(pad, dtype=jnp.int32) % (NPAD - N))
    dst_p = jnp.concatenate([dst, pad_dst])
    zeros = jnp.zeros((N, D), jnp.float32)

    p = _segment_sum(x, zeros, src_p, dst_p)
    h1 = _mlp(p[0], p[1], W1a, b1a, W1b, b1b)
    q = _segment_sum(h1, zeros, src_p, dst_p)
    h2 = _mlp(q[0], q[1], W2a, b2a, W2b, b2b)
    return h2


# speedup vs baseline: 1.1478x; 1.1478x over previous
import functools

import jax
import jax.numpy as jnp
from jax import lax
from jax.experimental import pallas as pl
from jax.experimental.pallas import tpu as pltpu
from jax.experimental.pallas import tpu_sc as plsc

N = 10000
E = 320000
D = 128

NC = 2
NS = 16
NW = NC * NS
C = 128
STEPS0 = 124
STEPS1 = 36
EP0 = STEPS0 * C
EP1 = STEPS1 * C
E_PAD = (EP0 + EP1) * NS
NPAD = 10240
RT = NPAD // NS
RT_LAST = N - (NS - 1) * RT

_mesh = plsc.VectorSubcoreMesh(core_axis_name="c", subcore_axis_name="s")


@functools.partial(
    pl.kernel,
    out_type=jax.ShapeDtypeStruct((NC, NPAD, D), jnp.float32),
    mesh=_mesh,
    scratch_types=[
        pltpu.VMEM((max(EP0, EP1),), jnp.int32),
        pltpu.VMEM((C,), jnp.int32),
        pltpu.VMEM((C,), jnp.int32),
        pltpu.VMEM((C, D), jnp.float32),
        pltpu.VMEM((C, D), jnp.float32),
        pltpu.VMEM_SHARED((NPAD, D), jnp.float32),
        pltpu.SemaphoreType.DMA,
        pltpu.SemaphoreType.DMA,
    ],
)
def _segment_sum(h_hbm, zeros_hbm, src_hbm, dst_hbm, out_hbm,
                 src_all, dst_v0, dst_v1, rows_v0, rows_v1, acc_sh,
                 sem0, sem1):
    cid = lax.axis_index("c")
    sid = lax.axis_index("s")
    last = sid == NS - 1
    r0 = sid * RT

    @pl.when(jnp.logical_and(cid == 0, jnp.logical_not(last)))
    def _():
        pltpu.sync_copy(h_hbm.at[pl.ds(r0, RT)], acc_sh.at[pl.ds(r0, RT)])

    @pl.when(jnp.logical_and(cid == 0, last))
    def _():
        pltpu.sync_copy(h_hbm.at[pl.ds(r0, RT_LAST)],
                        acc_sh.at[pl.ds(r0, RT_LAST)])

    @pl.when(jnp.logical_and(cid != 0, jnp.logical_not(last)))
    def _():
        pltpu.sync_copy(zeros_hbm.at[pl.ds(r0, RT)], acc_sh.at[pl.ds(r0, RT)])

    @pl.when(jnp.logical_and(cid != 0, last))
    def _():
        pltpu.sync_copy(zeros_hbm.at[pl.ds(r0, RT_LAST)],
                        acc_sh.at[pl.ds(r0, RT_LAST)])

    plsc.subcore_barrier()

    steps = jnp.where(cid == 0, STEPS0, STEPS1)
    base = jnp.where(cid == 0, sid * EP0, NS * EP0 + sid * EP1)
    dst_bufs = (dst_v0, dst_v1)
    row_bufs = (rows_v0, rows_v1)
    sems = (sem0, sem1)

    @pl.when(cid == 0)
    def _():
        pltpu.sync_copy(src_hbm.at[pl.ds(base, EP0)],
                        src_all.at[pl.ds(0, EP0)])

    @pl.when(cid != 0)
    def _():
        pltpu.sync_copy(src_hbm.at[pl.ds(base, EP1)],
                        src_all.at[pl.ds(0, EP1)])

    pltpu.async_copy(h_hbm.at[src_all.at[pl.ds(0, C)]], rows_v0, sem0)
    pltpu.async_copy(h_hbm.at[src_all.at[pl.ds(C, C)]], rows_v1, sem1)

    def body(k, carry):
        for t in range(2):
            j = 2 * k + t
            db, rb, sm = dst_bufs[t], row_bufs[t], sems[t]
            pltpu.make_async_copy(h_hbm.at[src_all.at[pl.ds(j * C, C)]],
                                  rb, sm).wait()
            pltpu.sync_copy(dst_hbm.at[pl.ds(base + j * C, C)], db)
            pltpu.sync_copy(rb, acc_sh.at[db], add=True)

            @pl.when(j + 2 < steps)
            def _():
                pltpu.async_copy(
                    h_hbm.at[src_all.at[pl.ds((j + 2) * C, C)]], rb, sm)
        return carry

    lax.fori_loop(0, steps // 2, body, 0)

    plsc.subcore_barrier()

    @pl.when(jnp.logical_not(last))
    def _():
        pltpu.sync_copy(acc_sh.at[pl.ds(r0, RT)],
                        out_hbm.at[cid, pl.ds(r0, RT)])

    @pl.when(last)
    def _():
        pltpu.sync_copy(acc_sh.at[pl.ds(r0, RT_LAST)],
                        out_hbm.at[cid, pl.ds(r0, RT_LAST)])


BN = 1000


def _mlp_body(p0_ref, p1_ref, wa_ref, ba_ref, wb_ref, bb_ref, o_ref):
    h = p0_ref[...] + p1_ref[...]
    a = jnp.maximum(
        jnp.dot(h, wa_ref[...], preferred_element_type=jnp.float32)
        + ba_ref[...], 0.0)
    o = jnp.maximum(
        jnp.dot(a, wb_ref[...], preferred_element_type=jnp.float32)
        + bb_ref[...], 0.0)
    o_ref[...] = o


def _mlp(p0, p1, Wa, ba, Wb, bb):
    return pl.pallas_call(
        _mlp_body,
        grid=(N // BN,),
        in_specs=[
            pl.BlockSpec((BN, D), lambda i: (i, 0)),
            pl.BlockSpec((BN, D), lambda i: (i, 0)),
            pl.BlockSpec((D, D), lambda i: (0, 0)),
            pl.BlockSpec((1, D), lambda i: (0, 0)),
            pl.BlockSpec((D, D), lambda i: (0, 0)),
            pl.BlockSpec((1, D), lambda i: (0, 0)),
        ],
        out_specs=pl.BlockSpec((BN, D), lambda i: (i, 0)),
        out_shape=jax.ShapeDtypeStruct((N, D), jnp.float32),
    )(p0, p1, Wa, ba.reshape(1, D), Wb, bb.reshape(1, D))


def kernel(x, edge_index, W1a, b1a, W1b, b1b, W2a, b2a, W2b, b2b):
    src = edge_index[0].astype(jnp.int32)
    dst = edge_index[1].astype(jnp.int32)
    pad = E_PAD - E
    src_p = jnp.concatenate([src, jnp.zeros((pad,), jnp.int32)])
    pad_dst = N + (jnp.arange(pad, dtype=jnp.int32) % (NPAD - N))
    dst_p = jnp.concatenate([dst, pad_dst])
    zeros = jnp.zeros((N, D), jnp.float32)

    p = _segment_sum(x, zeros, src_p, dst_p)
    h1 = _mlp(p[0], p[1], W1a, b1a, W1b, b1b)
    q = _segment_sum(h1, zeros, src_p, dst_p)
    h2 = _mlp(q[0], q[1], W2a, b2a, W2b, b2b)
    return h2
